# Initial kernel scaffold; baseline (speedup 1.0000x reference)
import jax
import jax.numpy as jnp
from jax.experimental import pallas as pl


def kernel(x, edge_index, edge_weight, W):
    raise NotImplementedError("write your pallas kernel here")


# speedup vs baseline: 4.4337x; 4.4337x over previous
import functools

import jax
import jax.numpy as jnp
from jax import lax
from jax.experimental import pallas as pl
from jax.experimental.pallas import tpu as pltpu
from jax.experimental.pallas import tpu_sc as plsc

N = 10000
D = 128
E = 320000
NC = 2
NS = 16
NW = NC * NS
EPW = E // NW
CH = 80
NCHUNK = EPW // CH
NPAD = 10240
RPT = NPAD // NS
ZR = 80
NV = D // 16

@functools.cache
def _make_sc_aggregate():
  mesh = plsc.VectorSubcoreMesh(
      core_axis_name="c", subcore_axis_name="s", num_cores=NC, num_subcores=NS)

  @functools.partial(
      pl.kernel,
      out_type=jax.ShapeDtypeStruct((NC, N, D), jnp.float32),
      mesh=mesh,
      scratch_types=[
          pltpu.MemorySpace.VMEM_SHARED((NPAD, D), jnp.float32),
          pltpu.VMEM((CH,), jnp.int32),
          pltpu.VMEM((CH,), jnp.int32),
          pltpu.VMEM((CH,), jnp.float32),
          pltpu.VMEM((CH, D), jnp.float32),
          pltpu.VMEM((ZR, D), jnp.float32),
          pltpu.SemaphoreType.DMA,
      ],
  )
  def _sc_aggregate(x_hbm, src_hbm, dst_hbm, w_hbm, out_hbm,
                    acc, src_v, dst_v, w_v, rows_v, stg_v, sem):
    c = lax.axis_index("c")
    s = lax.axis_index("s")
    wid = s * NC + c

    def zero_stage(i, _):
        for j in range(NV):
            stg_v[i, pl.ds(j * 16, 16)] = jnp.zeros((16,), jnp.float32)
        return 0
    lax.fori_loop(0, ZR, zero_stage, 0)

    def zero_acc(t, _):
        pltpu.sync_copy(stg_v, acc.at[pl.ds(s * RPT + t * ZR, ZR)])
        return 0
    lax.fori_loop(0, RPT // ZR, zero_acc, 0)
    plsc.subcore_barrier()

    base0 = wid * EPW

    def chunk(k, _):
        b = base0 + k * CH
        pltpu.sync_copy(src_hbm.at[pl.ds(b, CH)], src_v)
        pltpu.sync_copy(dst_hbm.at[pl.ds(b, CH)], dst_v)
        pltpu.sync_copy(w_hbm.at[pl.ds(b, CH)], w_v)
        pltpu.async_copy(x_hbm.at[src_v], rows_v, sem).wait()

        def scale(g, _):
            wv = w_v[pl.ds(g * 16, 16)]
            for l in range(16):
                wl = wv[l]
                e = g * 16 + l
                for j in range(NV):
                    sl = pl.ds(j * 16, 16)
                    rows_v[e, sl] = rows_v[e, sl] * wl
            return 0
        lax.fori_loop(0, CH // 16, scale, 0)

        pltpu.sync_copy(rows_v, acc.at[dst_v], add=True)
        return 0
    lax.fori_loop(0, NCHUNK, chunk, 0)
    plsc.subcore_barrier()

    def copy_out(t, _):
        r = s * RPT + t * ZR
        pltpu.sync_copy(acc.at[pl.ds(r, ZR)], stg_v)
        pltpu.sync_copy(stg_v, out_hbm.at[c, pl.ds(r, ZR)])
        return 0
    n_out = jnp.where(s == NS - 1, (N - (NS - 1) * RPT) // ZR, RPT // ZR)
    lax.fori_loop(0, n_out, copy_out, 0)

  return _sc_aggregate


BLK = 1000


def _tc_matmul_body(p0_ref, p1_ref, w_ref, y_ref, stats_ref, acc_ref):
    i = pl.program_id(0)
    agg = p0_ref[...] + p1_ref[...]
    y = jnp.dot(agg, w_ref[...], preferred_element_type=jnp.float32)
    y_ref[...] = y

    @pl.when(i == 0)
    def _():
        acc_ref[...] = jnp.zeros_like(acc_ref)

    acc_ref[0:1, :] += jnp.sum(y, axis=0, keepdims=True)
    acc_ref[1:2, :] += jnp.sum(y * y, axis=0, keepdims=True)

    @pl.when(i == pl.num_programs(0) - 1)
    def _():
        stats_ref[...] = acc_ref[...]


def _tc_norm_body(y_ref, stats_ref, o_ref):
    y = y_ref[...]
    mean = stats_ref[0:1, :] * (1.0 / N)
    var = stats_ref[1:2, :] * (1.0 / N) - mean * mean
    z = jnp.maximum((y - mean) * lax.rsqrt(var + 0.001), 0.0)
    sq = jnp.sum(z * z)
    o_ref[...] = z * lax.rsqrt(jnp.maximum(sq, 1e-12))


def kernel(x, edge_index, edge_weight, W):
    src = edge_index[0]
    dst = edge_index[1]
    parts = _make_sc_aggregate()(x, src, dst, edge_weight)

    y, stats = pl.pallas_call(
        _tc_matmul_body,
        grid=(N // BLK,),
        in_specs=[
            pl.BlockSpec((BLK, D), lambda i: (i, 0)),
            pl.BlockSpec((BLK, D), lambda i: (i, 0)),
            pl.BlockSpec((D, D), lambda i: (0, 0)),
        ],
        out_specs=[
            pl.BlockSpec((BLK, D), lambda i: (i, 0)),
            pl.BlockSpec((2, D), lambda i: (0, 0)),
        ],
        out_shape=[
            jax.ShapeDtypeStruct((N, D), jnp.float32),
            jax.ShapeDtypeStruct((2, D), jnp.float32),
        ],
        scratch_shapes=[pltpu.VMEM((2, D), jnp.float32)],
    )(parts[0], parts[1], W)

    out = pl.pallas_call(
        _tc_norm_body,
        out_shape=jax.ShapeDtypeStruct((N, D), jnp.float32),
    )(y, stats)
    return out


# speedup vs baseline: 11.5943x; 2.6150x over previous
import functools

import jax
import jax.numpy as jnp
from jax import lax
from jax.experimental import pallas as pl
from jax.experimental.pallas import tpu as pltpu
from jax.experimental.pallas import tpu_sc as plsc

N = 10000
D = 128
E = 320000
NC = 2
NS = 16
NW = NC * NS
EPW = E // NW
CH = 80
NCHUNK = EPW // CH
NBUF = 4
IR = 12
NPAD = 10240
RPT = NPAD // NS
ZR = 80
NV = D // 16

@functools.cache
def _make_sc_aggregate():
  mesh = plsc.VectorSubcoreMesh(
      core_axis_name="c", subcore_axis_name="s", num_cores=NC, num_subcores=NS)

  @functools.partial(
      pl.kernel,
      out_type=jax.ShapeDtypeStruct((NC, N, D), jnp.float32),
      mesh=mesh,
      scratch_types=[
          pltpu.MemorySpace.VMEM_SHARED((NPAD, D), jnp.float32),
          pltpu.VMEM((IR, CH), jnp.int32),
          pltpu.VMEM((IR, CH), jnp.int32),
          pltpu.VMEM((IR, CH), jnp.float32),
          pltpu.VMEM((NBUF, CH, D), jnp.float32),
          [pltpu.SemaphoreType.DMA] * NBUF,
          [pltpu.SemaphoreType.DMA] * NBUF,
          [pltpu.SemaphoreType.DMA] * IR,
      ],
  )
  def _sc_aggregate(x_hbm, src_hbm, dst_hbm, w_hbm, out_hbm,
                    acc, src_v, dst_v, w_v, rows_v, gsems, ssems, isems):
    c = lax.axis_index("c")
    s = lax.axis_index("s")
    wid = s * NC + c
    base0 = wid * EPW

    def idx_copies(k, isl):
        b0 = base0 + k * CH
        return (
            (src_hbm.at[pl.ds(b0, CH)], src_v.at[isl]),
            (dst_hbm.at[pl.ds(b0, CH)], dst_v.at[isl]),
            (w_hbm.at[pl.ds(b0, CH)], w_v.at[isl]),
        )

    def fire_idx(k, isl):
        for sref, dref in idx_copies(k, isl):
            pltpu.async_copy(sref, dref, isems[isl])

    def wait_idx(k, isl):
        for sref, dref in idx_copies(k, isl):
            pltpu.make_async_copy(sref, dref, isems[isl]).wait()

    def fire_gather(k, b, isl):
        pltpu.async_copy(x_hbm.at[src_v.at[isl]], rows_v.at[b], gsems[b])

    def wait_gather(k, b, isl):
        pltpu.make_async_copy(x_hbm.at[src_v.at[isl]], rows_v.at[b],
                              gsems[b]).wait()

    def fire_scatter(k, b, isl):
        pltpu.async_copy(rows_v.at[b], acc.at[dst_v.at[isl]], ssems[b],
                         add=True)

    def wait_scatter(k, b, isl):
        pltpu.make_async_copy(rows_v.at[b], acc.at[dst_v.at[isl]],
                              ssems[b]).wait()

    def scale(b, isl):
        def body(g, _):
            wv = w_v[isl, pl.ds(g * 16, 16)]
            for l in range(16):
                wl = wv[l]
                e = g * 16 + l
                for j in range(NV):
                    sl = pl.ds(j * 16, 16)
                    rows_v[b, e, sl] = rows_v[b, e, sl] * wl
            return 0
        lax.fori_loop(0, CH // 16, body, 0)

    def zero_stage(i, _):
        for j in range(NV):
            rows_v[0, i, pl.ds(j * 16, 16)] = jnp.zeros((16,), jnp.float32)
        return 0
    lax.fori_loop(0, ZR, zero_stage, 0)

    def zero_acc(t, _):
        pltpu.sync_copy(rows_v.at[0], acc.at[pl.ds(s * RPT + t * ZR, ZR)])
        return 0
    lax.fori_loop(0, RPT // ZR, zero_acc, 0)

    for k in range(8):
        fire_idx(k, k)
    for k in range(3):
        wait_idx(k, k)
        fire_gather(k, k, k)
    plsc.subcore_barrier()

    def maybe(cond, fn):
        if isinstance(cond, bool):
            if cond:
                fn()
        else:
            pl.when(cond)(fn)

    def slot(kb, j, k):
        b = j % 4
        isl = j % IR
        wait_gather(k, b, isl)
        scale(b, isl)
        fire_scatter(k, b, isl)
        maybe(k + 8 < NCHUNK, lambda: fire_idx(k + 8, (j + 8) % IR))
        maybe(k > 0,
              lambda: wait_scatter(k - 1, (b + 3) % 4, (j + 11) % IR))

        def _advance():
            wait_idx(k + 3, (j + 3) % IR)
            fire_gather(k + 3, (b + 3) % 4, (j + 3) % IR)
        maybe(k + 3 < NCHUNK, _advance)

    def outer(kb, _):
        for j in range(IR):
            slot(kb, j, kb * IR + j)
        return 0
    NMAIN = (NCHUNK // IR) * IR
    lax.fori_loop(0, NCHUNK // IR, outer, 0)
    for t in range(NCHUNK - NMAIN):
        slot(NCHUNK // IR, t, NMAIN + t)
    wait_scatter(NCHUNK - 1, (NCHUNK - 1) % 4, (NCHUNK - 1) % IR)
    plsc.subcore_barrier()

    def copy_out(t, _):
        r = s * RPT + t * ZR
        pltpu.sync_copy(acc.at[pl.ds(r, ZR)], rows_v.at[0])
        pltpu.sync_copy(rows_v.at[0], out_hbm.at[c, pl.ds(r, ZR)])
        return 0
    n_out = jnp.where(s == NS - 1, (N - (NS - 1) * RPT) // ZR, RPT // ZR)
    lax.fori_loop(0, n_out, copy_out, 0)

  return _sc_aggregate


BLK = 1000


def _tc_matmul_body(p0_ref, p1_ref, w_ref, y_ref, stats_ref, acc_ref):
    i = pl.program_id(0)
    agg = p0_ref[...] + p1_ref[...]
    y = jnp.dot(agg, w_ref[...], preferred_element_type=jnp.float32)
    y_ref[...] = y

    @pl.when(i == 0)
    def _():
        acc_ref[...] = jnp.zeros_like(acc_ref)

    acc_ref[0:1, :] += jnp.sum(y, axis=0, keepdims=True)
    acc_ref[1:2, :] += jnp.sum(y * y, axis=0, keepdims=True)

    @pl.when(i == pl.num_programs(0) - 1)
    def _():
        stats_ref[...] = acc_ref[...]


def _tc_norm_body(y_ref, stats_ref, o_ref):
    y = y_ref[...]
    mean = stats_ref[0:1, :] * (1.0 / N)
    var = stats_ref[1:2, :] * (1.0 / N) - mean * mean
    z = jnp.maximum((y - mean) * lax.rsqrt(var + 0.001), 0.0)
    sq = jnp.sum(z * z)
    o_ref[...] = z * lax.rsqrt(jnp.maximum(sq, 1e-12))


def kernel(x, edge_index, edge_weight, W):
    src = edge_index[0]
    dst = edge_index[1]
    parts = _make_sc_aggregate()(x, src, dst, edge_weight)

    y, stats = pl.pallas_call(
        _tc_matmul_body,
        grid=(N // BLK,),
        in_specs=[
            pl.BlockSpec((BLK, D), lambda i: (i, 0)),
            pl.BlockSpec((BLK, D), lambda i: (i, 0)),
            pl.BlockSpec((D, D), lambda i: (0, 0)),
        ],
        out_specs=[
            pl.BlockSpec((BLK, D), lambda i: (i, 0)),
            pl.BlockSpec((2, D), lambda i: (0, 0)),
        ],
        out_shape=[
            jax.ShapeDtypeStruct((N, D), jnp.float32),
            jax.ShapeDtypeStruct((2, D), jnp.float32),
        ],
        scratch_shapes=[pltpu.VMEM((2, D), jnp.float32)],
    )(parts[0], parts[1], W)

    out = pl.pallas_call(
        _tc_norm_body,
        out_shape=jax.ShapeDtypeStruct((N, D), jnp.float32),
    )(y, stats)
    return out


# speedup vs baseline: 12.6376x; 1.0900x over previous
import functools

import jax
import jax.numpy as jnp
from jax import lax
from jax.experimental import pallas as pl
from jax.experimental.pallas import tpu as pltpu
from jax.experimental.pallas import tpu_sc as plsc

N = 10000
D = 128
E = 320000
NC = 2
NS = 16
NW = NC * NS
EPW = E // NW
CH = 80
NCHUNK = EPW // CH
NBUF = 4
IR = 12
NPAD = 10240
RPT = NPAD // NS
ZR = 80
NV = D // 16

@functools.cache
def _make_sc_aggregate():
  mesh = plsc.VectorSubcoreMesh(
      core_axis_name="c", subcore_axis_name="s", num_cores=NC, num_subcores=NS)

  @functools.partial(
      pl.kernel,
      out_type=jax.ShapeDtypeStruct((NC, N, D), jnp.float32),
      mesh=mesh,
      scratch_types=[
          pltpu.MemorySpace.VMEM_SHARED((NPAD, D), jnp.float32),
          pltpu.VMEM((IR, CH), jnp.int32),
          pltpu.VMEM((IR, CH), jnp.int32),
          pltpu.VMEM((IR, CH), jnp.float32),
          pltpu.VMEM((NBUF, CH, D), jnp.float32),
          [pltpu.SemaphoreType.DMA] * NBUF,
          [pltpu.SemaphoreType.DMA] * NBUF,
          [pltpu.SemaphoreType.DMA] * IR,
      ],
  )
  def _sc_aggregate(x_hbm, src_hbm, dst_hbm, w_hbm, out_hbm,
                    acc, src_v, dst_v, w_v, rows_v, gsems, ssems, isems):
    c = lax.axis_index("c")
    s = lax.axis_index("s")
    wid = s * NC + c
    base0 = wid * EPW

    def idx_copies(k, isl):
        b0 = base0 + k * CH
        return (
            (src_hbm.at[pl.ds(b0, CH)], src_v.at[isl]),
            (dst_hbm.at[pl.ds(b0, CH)], dst_v.at[isl]),
            (w_hbm.at[pl.ds(b0, CH)], w_v.at[isl]),
        )

    def fire_idx(k, isl):
        for sref, dref in idx_copies(k, isl):
            pltpu.async_copy(sref, dref, isems[isl])

    def wait_idx(k, isl):
        for sref, dref in idx_copies(k, isl):
            pltpu.make_async_copy(sref, dref, isems[isl]).wait()

    def fire_gather(k, b, isl):
        pltpu.async_copy(x_hbm.at[src_v.at[isl]], rows_v.at[b], gsems[b])

    def wait_gather(k, b, isl):
        pltpu.make_async_copy(x_hbm.at[src_v.at[isl]], rows_v.at[b],
                              gsems[b]).wait()

    def fire_scatter(k, b, isl):
        pltpu.async_copy(rows_v.at[b], acc.at[dst_v.at[isl]], ssems[b],
                         add=True)

    def wait_scatter(k, b, isl):
        pltpu.make_async_copy(rows_v.at[b], acc.at[dst_v.at[isl]],
                              ssems[b]).wait()

    def scale(b, isl):
        def body(g, _):
            wv = w_v[isl, pl.ds(g * 16, 16)]
            for l in range(16):
                wl = wv[l]
                e = g * 16 + l
                for j in range(NV):
                    sl = pl.ds(j * 16, 16)
                    rows_v[b, e, sl] = rows_v[b, e, sl] * wl
            return 0
        lax.fori_loop(0, CH // 16, body, 0)

    def zero_stage(i, _):
        for j in range(NV):
            rows_v[0, i, pl.ds(j * 16, 16)] = jnp.zeros((16,), jnp.float32)
        return 0
    lax.fori_loop(0, ZR, zero_stage, 0)

    def zero_acc(t, _):
        pltpu.sync_copy(rows_v.at[0], acc.at[pl.ds(s * RPT + t * ZR, ZR)])
        return 0
    lax.fori_loop(0, RPT // ZR, zero_acc, 0)

    for k in range(8):
        fire_idx(k, k)
    for k in range(3):
        wait_idx(k, k)
        fire_gather(k, k, k)
    plsc.subcore_barrier()

    def maybe(cond, fn):
        if isinstance(cond, bool):
            if cond:
                fn()
        else:
            pl.when(cond)(fn)

    def slot(kb, j, k):
        b = j % 4
        isl = j % IR
        wait_gather(k, b, isl)
        scale(b, isl)
        fire_scatter(k, b, isl)
        maybe(k + 8 < NCHUNK, lambda: fire_idx(k + 8, (j + 8) % IR))
        maybe(k > 0,
              lambda: wait_scatter(k - 1, (b + 3) % 4, (j + 11) % IR))

        def _advance():
            wait_idx(k + 3, (j + 3) % IR)
            fire_gather(k + 3, (b + 3) % 4, (j + 3) % IR)
        maybe(k + 3 < NCHUNK, _advance)

    def outer(kb, _):
        for j in range(IR):
            slot(kb, j, kb * IR + j)
        return 0
    NMAIN = (NCHUNK // IR) * IR
    lax.fori_loop(0, NCHUNK // IR, outer, 0)
    for t in range(NCHUNK - NMAIN):
        slot(NCHUNK // IR, t, NMAIN + t)
    wait_scatter(NCHUNK - 1, (NCHUNK - 1) % 4, (NCHUNK - 1) % IR)
    plsc.subcore_barrier()

    def copy_out(t, _):
        r = s * RPT + t * ZR
        pltpu.sync_copy(acc.at[pl.ds(r, ZR)], rows_v.at[0])
        pltpu.sync_copy(rows_v.at[0], out_hbm.at[c, pl.ds(r, ZR)])
        return 0
    n_out = jnp.where(s == NS - 1, (N - (NS - 1) * RPT) // ZR, RPT // ZR)
    lax.fori_loop(0, n_out, copy_out, 0)

  return _sc_aggregate


def _tc_body(p_ref, w_ref, o_ref):
    agg = p_ref[0] + p_ref[1]
    y = jnp.dot(agg, w_ref[...], preferred_element_type=jnp.float32)
    mean = jnp.mean(y, axis=0, keepdims=True)
    var = jnp.mean(y * y, axis=0, keepdims=True) - mean * mean
    z = jnp.maximum((y - mean) * lax.rsqrt(var + 0.001), 0.0)
    sq = jnp.sum(z * z)
    o_ref[...] = z * lax.rsqrt(jnp.maximum(sq, 1e-12))


def kernel(x, edge_index, edge_weight, W):
    src = edge_index[0]
    dst = edge_index[1]
    parts = _make_sc_aggregate()(x, src, dst, edge_weight)
    out = pl.pallas_call(
        _tc_body,
        out_shape=jax.ShapeDtypeStruct((N, D), jnp.float32),
    )(parts, W)
    return out
